# Initial kernel scaffold; baseline (speedup 1.0000x reference)
import jax
import jax.numpy as jnp
from jax.experimental import pallas as pl


def kernel(images, features, segment_map):
    raise NotImplementedError("write your pallas kernel here")


# speedup vs baseline: 4.2168x; 4.2168x over previous
import functools

import jax
import jax.numpy as jnp
from jax import lax
from jax.experimental import pallas as pl
from jax.experimental.pallas import tpu as pltpu
from jax.experimental.pallas import tpu_sc as plsc

N_SEG = 2048
L = 16


@functools.lru_cache(maxsize=None)
def _build_sc_kernel(B, F, P, CPW, CHUNK):
    NCHUNK = P // CHUNK
    GROUPS = CHUNK // L
    WPB = F // CPW
    assert B * WPB == 32, "mapping assumes 32 SC subcores"

    mesh = plsc.VectorSubcoreMesh(core_axis_name="c", subcore_axis_name="s")

    @functools.partial(
        pl.kernel,
        mesh=mesh,
        compiler_params=pltpu.CompilerParams(needs_layout_passes=False),
        out_type=[
            jax.ShapeDtypeStruct((B * F * N_SEG,), jnp.float32),
            jax.ShapeDtypeStruct((B * N_SEG,), jnp.float32),
            jax.ShapeDtypeStruct((B * 2 * N_SEG,), jnp.float32),
        ],
        scratch_types=[
            pltpu.VMEM((CHUNK,), jnp.int32),
            pltpu.VMEM((CPW, CHUNK), jnp.float32),
            pltpu.VMEM((CPW * N_SEG,), jnp.float32),
            pltpu.VMEM((N_SEG,), jnp.float32),
            pltpu.VMEM((N_SEG,), jnp.float32),
            pltpu.VMEM((N_SEG,), jnp.float32),
            pltpu.SemaphoreType.DMA,
        ],
    )
    def sp_kernel(feat_hbm, seg_hbm, tok_hbm, cnt_hbm, cent_hbm,
                  seg_v, feat_v, acc_v, cnt_v, cent_v, recip_v, dma_sem):
        cid = lax.axis_index("c")
        sid = lax.axis_index("s")
        batch = cid * (B // 2) + sid // WPB
        wib = sid % WPB
        c0 = wib * CPW

        iota = lax.iota(jnp.int32, L)
        zeros = jnp.zeros((L,), jnp.float32)
        ones = jnp.ones((L,), jnp.float32)
        col_base = [jnp.full((L,), cc * N_SEG, jnp.int32) for cc in range(CPW)]

        def zero_small(i, _):
            cnt_v[pl.ds(i * L, L)] = zeros
            cent_v[pl.ds(i * L, L)] = zeros
            return 0
        lax.fori_loop(0, N_SEG // L, zero_small, 0)

        def zero_acc(i, _):
            acc_v[pl.ds(i * L, L)] = zeros
            return 0
        lax.fori_loop(0, (CPW * N_SEG) // L, zero_acc, 0)

        feat_base = (batch * F + c0) * P

        def chunk_body(g, _):
            base_px = g * CHUNK
            copies = [pltpu.async_copy(
                seg_hbm.at[pl.ds(batch * P + base_px, CHUNK)], seg_v, dma_sem)]
            for cc in range(CPW):
                copies.append(pltpu.async_copy(
                    feat_hbm.at[pl.ds(feat_base + cc * P + base_px, CHUNK)],
                    feat_v.at[cc], dma_sem))
            for cp in copies:
                cp.wait()

            def grp(i, _):
                off = i * L
                seg16 = seg_v[pl.ds(off, L)]
                for cc in range(CPW):
                    val = feat_v[cc, pl.ds(off, L)]
                    plsc.addupdate_scatter(acc_v, [seg16 + col_base[cc]], val)
                plsc.addupdate_scatter(cnt_v, [seg16], ones)
                return 0
            lax.fori_loop(0, GROUPS, grp, 0)

            @pl.when(jnp.logical_or(wib == 1, wib == 2))
            def _():
                is_x = jnp.full((L,), wib) == 1

                def grpc(i, _):
                    off = i * L
                    seg16 = seg_v[pl.ds(off, L)]
                    p = base_px + off + iota
                    x = (p & 511).astype(jnp.float32)
                    y = (p >> 9).astype(jnp.float32)
                    v = jnp.where(is_x, x, y) * (1.0 / 511.0)
                    plsc.addupdate_scatter(cent_v, [seg16], v)
                    return 0
                lax.fori_loop(0, GROUPS, grpc, 0)
            return 0
        lax.fori_loop(0, NCHUNK, chunk_body, 0)

        def rec(i, _):
            c16 = cnt_v[pl.ds(i * L, L)]
            recip_v[pl.ds(i * L, L)] = 1.0 / jnp.maximum(c16, 1.0)
            return 0
        lax.fori_loop(0, N_SEG // L, rec, 0)

        def div_acc(i, _):
            s = pl.ds(i * L, L)
            r = pl.ds((i * L) & (N_SEG - 1), L)
            acc_v[s] = acc_v[s] * recip_v[r]
            return 0
        lax.fori_loop(0, (CPW * N_SEG) // L, div_acc, 0)

        pltpu.sync_copy(
            acc_v, tok_hbm.at[pl.ds((batch * F + c0) * N_SEG, CPW * N_SEG)])

        @pl.when(wib == 0)
        def _():
            pltpu.sync_copy(cnt_v, cnt_hbm.at[pl.ds(batch * N_SEG, N_SEG)])

        @pl.when(jnp.logical_or(wib == 1, wib == 2))
        def _():
            def div_cent(i, _):
                s = pl.ds(i * L, L)
                cent_v[s] = cent_v[s] * recip_v[s]
                return 0
            lax.fori_loop(0, N_SEG // L, div_cent, 0)
            pltpu.sync_copy(
                cent_v,
                cent_hbm.at[pl.ds((batch * 2 + (wib - 1)) * N_SEG, N_SEG)])

    return sp_kernel


def kernel(images, features, segment_map):
    B, F, H, W = features.shape
    P = H * W
    feats = features.reshape(B * F * P)
    segs = segment_map.reshape(B * P)
    tok1d, cnt1d, cent1d = _build_sc_kernel(B, F, P, 12, 2048)(feats, segs)
    tokens = tok1d.reshape(B, F, N_SEG).transpose(0, 2, 1)
    counts = cnt1d.reshape(B, N_SEG)
    centroids = cent1d.reshape(B, 2, N_SEG).transpose(0, 2, 1)
    attention_mask = counts > 0
    return (tokens, segment_map, attention_mask, centroids)


# speedup vs baseline: 4.5110x; 1.0698x over previous
import functools

import jax
import jax.numpy as jnp
from jax import lax
from jax.experimental import pallas as pl
from jax.experimental.pallas import tpu as pltpu
from jax.experimental.pallas import tpu_sc as plsc

N_SEG = 2048
L = 16


@functools.lru_cache(maxsize=None)
def _build_sc_kernel(B, F, P, CPW, CHUNK):
    NCHUNK = P // CHUNK
    GROUPS = CHUNK // L
    WPB = F // CPW
    assert B * WPB == 32, "mapping assumes 32 SC subcores"

    mesh = plsc.VectorSubcoreMesh(core_axis_name="c", subcore_axis_name="s")

    @functools.partial(
        pl.kernel,
        mesh=mesh,
        compiler_params=pltpu.CompilerParams(needs_layout_passes=False),
        out_type=[
            jax.ShapeDtypeStruct((B * F * N_SEG,), jnp.float32),
            jax.ShapeDtypeStruct((B * N_SEG,), jnp.float32),
            jax.ShapeDtypeStruct((B * 2 * N_SEG,), jnp.float32),
        ],
        scratch_types=[
            pltpu.VMEM((2, CHUNK), jnp.int32),
            pltpu.VMEM((2, CPW, CHUNK), jnp.float32),
            pltpu.VMEM((CPW * N_SEG,), jnp.float32),
            pltpu.VMEM((N_SEG,), jnp.float32),
            pltpu.VMEM((N_SEG,), jnp.float32),
            pltpu.VMEM((N_SEG,), jnp.float32),
            pltpu.SemaphoreType.DMA,
            pltpu.SemaphoreType.DMA,
        ],
    )
    def sp_kernel(feat_hbm, seg_hbm, tok_hbm, cnt_hbm, cent_hbm,
                  seg_v, feat_v, acc_v, cnt_v, cent_v, recip_v,
                  sem0, sem1):
        cid = lax.axis_index("c")
        sid = lax.axis_index("s")
        batch = cid * (B // 2) + sid // WPB
        wib = sid % WPB
        c0 = wib * CPW

        iota = lax.iota(jnp.int32, L)
        zeros = jnp.zeros((L,), jnp.float32)
        ones = jnp.ones((L,), jnp.float32)
        col_base = [jnp.full((L,), cc * N_SEG, jnp.int32) for cc in range(CPW)]

        def zero_small(i, _):
            cnt_v[pl.ds(i * L, L)] = zeros
            cent_v[pl.ds(i * L, L)] = zeros
            return 0
        lax.fori_loop(0, N_SEG // L, zero_small, 0)

        def zero_acc(i, _):
            acc_v[pl.ds(i * L, L)] = zeros
            return 0
        lax.fori_loop(0, (CPW * N_SEG) // L, zero_acc, 0)

        feat_base = (batch * F + c0) * P
        sems = [sem0, sem1]

        def copies(g, sl):
            base_px = g * CHUNK
            cps = [pltpu.make_async_copy(
                seg_hbm.at[pl.ds(batch * P + base_px, CHUNK)],
                seg_v.at[sl], sems[sl])]
            for cc in range(CPW):
                cps.append(pltpu.make_async_copy(
                    feat_hbm.at[pl.ds(feat_base + cc * P + base_px, CHUNK)],
                    feat_v.at[sl, cc], sems[sl]))
            return cps

        def issue(g, sl):
            for cp in copies(g, sl):
                cp.start()

        def drain(g, sl):
            for cp in copies(g, sl):
                cp.wait()

        issue(0, 0)
        issue(1, 1)

        def process(g, sl, base_px):
            def grp(i, _):
                off = i * L
                seg16 = seg_v[sl, pl.ds(off, L)]
                for cc in range(CPW):
                    val = feat_v[sl, cc, pl.ds(off, L)]
                    plsc.addupdate_scatter(acc_v, [seg16 + col_base[cc]], val)
                plsc.addupdate_scatter(cnt_v, [seg16], ones)
                return 0
            lax.fori_loop(0, GROUPS, grp, 0)

            @pl.when(jnp.logical_or(wib == 1, wib == 2))
            def _():
                is_x = jnp.full((L,), wib) == 1

                def grpc(i, _):
                    off = i * L
                    seg16 = seg_v[sl, pl.ds(off, L)]
                    p = base_px + off + iota
                    x = (p & 511).astype(jnp.float32)
                    y = (p >> 9).astype(jnp.float32)
                    v = jnp.where(is_x, x, y) * (1.0 / 511.0)
                    plsc.addupdate_scatter(cent_v, [seg16], v)
                    return 0
                lax.fori_loop(0, GROUPS, grpc, 0)

        def outer(gg, _):
            for sl in range(2):
                g = gg * 2 + sl
                drain(g, sl)
                process(g, sl, g * CHUNK)

                @pl.when(g + 2 < NCHUNK)
                def _():
                    issue(g + 2, sl)
            return 0
        lax.fori_loop(0, NCHUNK // 2, outer, 0)

        def rec(i, _):
            c16 = cnt_v[pl.ds(i * L, L)]
            recip_v[pl.ds(i * L, L)] = 1.0 / jnp.maximum(c16, 1.0)
            return 0
        lax.fori_loop(0, N_SEG // L, rec, 0)

        def div_acc(i, _):
            s = pl.ds(i * L, L)
            r = pl.ds((i * L) & (N_SEG - 1), L)
            acc_v[s] = acc_v[s] * recip_v[r]
            return 0
        lax.fori_loop(0, (CPW * N_SEG) // L, div_acc, 0)

        pltpu.sync_copy(
            acc_v, tok_hbm.at[pl.ds((batch * F + c0) * N_SEG, CPW * N_SEG)])

        @pl.when(wib == 0)
        def _():
            pltpu.sync_copy(cnt_v, cnt_hbm.at[pl.ds(batch * N_SEG, N_SEG)])

        @pl.when(jnp.logical_or(wib == 1, wib == 2))
        def _():
            def div_cent(i, _):
                s = pl.ds(i * L, L)
                cent_v[s] = cent_v[s] * recip_v[s]
                return 0
            lax.fori_loop(0, N_SEG // L, div_cent, 0)
            pltpu.sync_copy(
                cent_v,
                cent_hbm.at[pl.ds((batch * 2 + (wib - 1)) * N_SEG, N_SEG)])

    return sp_kernel


def kernel(images, features, segment_map):
    B, F, H, W = features.shape
    P = H * W
    feats = features.reshape(B * F * P)
    segs = segment_map.reshape(B * P)
    tok1d, cnt1d, cent1d = _build_sc_kernel(B, F, P, 12, 2048)(feats, segs)
    tokens = tok1d.reshape(B, F, N_SEG).transpose(0, 2, 1)
    counts = cnt1d.reshape(B, N_SEG)
    centroids = cent1d.reshape(B, 2, N_SEG).transpose(0, 2, 1)
    attention_mask = counts > 0
    return (tokens, segment_map, attention_mask, centroids)


# speedup vs baseline: 7.2465x; 1.6064x over previous
import functools

import jax
import jax.numpy as jnp
from jax import lax
from jax.experimental import pallas as pl
from jax.experimental.pallas import tpu as pltpu
from jax.experimental.pallas import tpu_sc as plsc

N_SEG = 2048
L = 16


@functools.lru_cache(maxsize=None)
def _build_sc_kernel(B, F, P, CPW, CHUNK):
    NCHUNK = P // CHUNK
    GROUPS = CHUNK // L
    WPB = F // CPW
    assert B * WPB == 32, "mapping assumes 32 SC subcores"

    mesh = plsc.VectorSubcoreMesh(core_axis_name="c", subcore_axis_name="s")

    @functools.partial(
        pl.kernel,
        mesh=mesh,
        compiler_params=pltpu.CompilerParams(needs_layout_passes=False),
        out_type=[
            jax.ShapeDtypeStruct((B * F * N_SEG,), jnp.float32),
            jax.ShapeDtypeStruct((B * N_SEG,), jnp.float32),
            jax.ShapeDtypeStruct((B * 2 * N_SEG,), jnp.float32),
        ],
        scratch_types=[
            pltpu.VMEM((2, CHUNK), jnp.int32),
            pltpu.VMEM((2, CPW, CHUNK), jnp.float32),
            pltpu.VMEM((CPW * N_SEG,), jnp.float32),
            pltpu.VMEM((N_SEG,), jnp.float32),
            pltpu.VMEM((N_SEG,), jnp.float32),
            pltpu.VMEM((N_SEG,), jnp.float32),
            pltpu.SemaphoreType.DMA,
            pltpu.SemaphoreType.DMA,
        ],
    )
    def sp_kernel(feat_hbm, seg_hbm, tok_hbm, cnt_hbm, cent_hbm,
                  seg_v, feat_v, acc_v, cnt_v, cent_v, recip_v,
                  sem0, sem1):
        cid = lax.axis_index("c")
        sid = lax.axis_index("s")
        batch = cid * (B // 2) + sid // WPB
        wib = sid % WPB
        c0 = wib * CPW

        iota = lax.iota(jnp.int32, L)
        zeros = jnp.zeros((L,), jnp.float32)
        ones = jnp.ones((L,), jnp.float32)
        col_base = [jnp.full((L,), cc * N_SEG, jnp.int32) for cc in range(CPW)]

        def zero_small(i, _):
            cnt_v[pl.ds(i * L, L)] = zeros
            cent_v[pl.ds(i * L, L)] = zeros
            return 0
        lax.fori_loop(0, N_SEG // L, zero_small, 0)

        @plsc.parallel_loop(0, (CPW * N_SEG) // L, unroll=8)
        def zero_acc(i):
            acc_v[pl.ds(i * L, L)] = zeros

        feat_base = (batch * F + c0) * P
        sems = [sem0, sem1]

        def copies(g, sl):
            base_px = g * CHUNK
            cps = [pltpu.make_async_copy(
                seg_hbm.at[pl.ds(batch * P + base_px, CHUNK)],
                seg_v.at[sl], sems[sl])]
            for cc in range(CPW):
                cps.append(pltpu.make_async_copy(
                    feat_hbm.at[pl.ds(feat_base + cc * P + base_px, CHUNK)],
                    feat_v.at[sl, cc], sems[sl]))
            return cps

        def issue(g, sl):
            for cp in copies(g, sl):
                cp.start()

        def drain(g, sl):
            for cp in copies(g, sl):
                cp.wait()

        issue(0, 0)
        issue(1, 1)

        def process(g, sl, base_px):
            @plsc.parallel_loop(0, GROUPS, unroll=4)
            def grp(i):
                off = i * L
                seg16 = seg_v[sl, pl.ds(off, L)]
                for cc in range(CPW):
                    val = feat_v[sl, cc, pl.ds(off, L)]
                    plsc.addupdate_scatter(acc_v, [seg16 + col_base[cc]], val)
                plsc.addupdate_scatter(cnt_v, [seg16], ones)

            @pl.when(jnp.logical_or(wib == 1, wib == 2))
            def _():
                is_x = jnp.full((L,), wib) == 1

                @plsc.parallel_loop(0, GROUPS, unroll=4)
                def grpc(i):
                    off = i * L
                    seg16 = seg_v[sl, pl.ds(off, L)]
                    p = base_px + off + iota
                    x = (p & 511).astype(jnp.float32)
                    y = (p >> 9).astype(jnp.float32)
                    v = jnp.where(is_x, x, y) * (1.0 / 511.0)
                    plsc.addupdate_scatter(cent_v, [seg16], v)

        def outer(gg, _):
            for sl in range(2):
                g = gg * 2 + sl
                drain(g, sl)
                process(g, sl, g * CHUNK)

                @pl.when(g + 2 < NCHUNK)
                def _():
                    issue(g + 2, sl)
            return 0
        lax.fori_loop(0, NCHUNK // 2, outer, 0)

        def rec(i, _):
            c16 = cnt_v[pl.ds(i * L, L)]
            recip_v[pl.ds(i * L, L)] = 1.0 / jnp.maximum(c16, 1.0)
            return 0
        lax.fori_loop(0, N_SEG // L, rec, 0)

        @plsc.parallel_loop(0, (CPW * N_SEG) // L, unroll=8)
        def div_acc(i):
            s = pl.ds(i * L, L)
            r = pl.ds((i * L) & (N_SEG - 1), L)
            acc_v[s] = acc_v[s] * recip_v[r]

        pltpu.sync_copy(
            acc_v, tok_hbm.at[pl.ds((batch * F + c0) * N_SEG, CPW * N_SEG)])

        @pl.when(wib == 0)
        def _():
            pltpu.sync_copy(cnt_v, cnt_hbm.at[pl.ds(batch * N_SEG, N_SEG)])

        @pl.when(jnp.logical_or(wib == 1, wib == 2))
        def _():
            def div_cent(i, _):
                s = pl.ds(i * L, L)
                cent_v[s] = cent_v[s] * recip_v[s]
                return 0
            lax.fori_loop(0, N_SEG // L, div_cent, 0)
            pltpu.sync_copy(
                cent_v,
                cent_hbm.at[pl.ds((batch * 2 + (wib - 1)) * N_SEG, N_SEG)])

    return sp_kernel


def kernel(images, features, segment_map):
    B, F, H, W = features.shape
    P = H * W
    feats = features.reshape(B * F * P)
    segs = segment_map.reshape(B * P)
    tok1d, cnt1d, cent1d = _build_sc_kernel(B, F, P, 12, 2048)(feats, segs)
    tokens = tok1d.reshape(B, F, N_SEG).transpose(0, 2, 1)
    counts = cnt1d.reshape(B, N_SEG)
    centroids = cent1d.reshape(B, 2, N_SEG).transpose(0, 2, 1)
    attention_mask = counts > 0
    return (tokens, segment_map, attention_mask, centroids)
